# baseline (speedup 1.0000x reference)
import functools
import jax
import jax.numpy as jnp
from jax import lax
from jax.experimental import pallas as pl
from jax.experimental.pallas import tpu as pltpu
from jax.experimental.pallas import tpu_sc as plsc

_GAMMA = 2.0
_ALPHA = 0.75
_NEG_RATIO = 3.0

_ROWS = 32
_N = 32768
_NC = 2
_NS = 16
_L = 16


def _prep_body(pred_ref, label_ref, key_ref, lw_ref, kb_ref, den_ref, posw_ref):
    pred = pred_ref[...]
    label = label_ref[...]
    n = pred.shape[1]

    e = jnp.exp(-jnp.abs(pred))
    log1pe = jnp.log(1.0 + e)
    ls_pos = jnp.minimum(pred, 0.0) - log1pe
    ls_neg = jnp.minimum(-pred, 0.0) - log1pe
    p = jnp.where(pred >= 0.0, 1.0 / (1.0 + e), e / (1.0 + e))

    loss = -(label * ls_pos + (1.0 - label) * ls_neg)
    p_t = label * p + (1.0 - label) * (1.0 - p)
    m = 1.0 - p_t
    loss = loss * (m * m)
    alpha_factor = label * _ALPHA + (1.0 - label) * (1.0 - _ALPHA)
    loss = loss * alpha_factor

    fn = (p < 0.5) & (label == 1.0)
    fp = (p >= 0.5) & (label == 0.0)
    w = _ALPHA / (1.0 - _ALPHA)
    loss_w = jnp.where(fn | fp, loss * w, loss)

    pos = label > 0.0
    num_pos = jnp.sum(pos.astype(jnp.int32), axis=1, keepdims=True)
    num_neg = (_NEG_RATIO * num_pos.astype(jnp.float32)).astype(jnp.int32)
    k = jnp.minimum(num_neg, n - num_pos)

    bits = lax.bitcast_convert_type(loss, jnp.int32)
    key_ref[...] = jnp.where(pos, 0, bits + 1)
    lw_ref[...] = loss_w
    kb_ref[...] = jnp.broadcast_to(k, (pred.shape[0], _L))
    den_ref[...] = jnp.broadcast_to(num_pos + k, (pred.shape[0], _L))
    posw_ref[...] = jnp.broadcast_to(
        jnp.sum(jnp.where(pos, loss_w, 0.0), axis=1, keepdims=True),
        (pred.shape[0], _L),
    )


def _sc_select_body(key_hbm, lw_hbm, kb_hbm, neg_hbm, key_v, lw_v, kv, hist, outv, sem):
    cid = lax.axis_index("c")
    sid = lax.axis_index("s")
    wid = sid * _NC + cid

    lw_cp = pltpu.async_copy(lw_hbm.at[wid], lw_v, sem)
    pltpu.sync_copy(key_hbm.at[wid], key_v)
    pltpu.sync_copy(kb_hbm.at[wid], kv)
    k = lax.shift_right_logical(jnp.sum(kv[...]), 4)

    outv[...] = jnp.zeros((_L,), jnp.float32)

    lane = lax.iota(jnp.int32, _L)
    ones = jnp.ones((_L,), jnp.int32)
    nchunks = _N // _L

    @pl.when(k > 0)
    def _():
        def radix_round(shift, prefix, kk):
            def zero_body(j, _):
                hist[pl.ds(j * _L, _L)] = jnp.zeros((_L,), jnp.int32)
                return 0

            lax.fori_loop(0, 4096 // _L, zero_body, 0)

            def sweep(i, _):
                v = key_v[pl.ds(i * _L, _L)]
                d = lax.shift_right_logical(v, shift) & 0xFF
                idx = lane * 256 + d
                if shift == 24:
                    plsc.addupdate_scatter(hist, [idx], ones)
                else:
                    m = lax.shift_right_logical(v, shift + 8) == prefix
                    plsc.addupdate_scatter(hist, [idx], ones, mask=m)
                return 0

            lax.fori_loop(0, nchunks, sweep, 0)

            def acc_chunk(c):
                def lane_body(l, a):
                    return a + hist[pl.ds((l * _L + c) * _L, _L)]

                return lax.fori_loop(0, _L, lane_body, jnp.zeros((_L,), jnp.int32))

            def chunk_body(j, carry):
                run, cc, run_c = carry
                c = 15 - j
                s_c = jnp.sum(acc_chunk(c))
                here = jnp.logical_and(run + s_c >= kk, cc < 0)
                cc = jnp.where(here, c, cc)
                run_c = jnp.where(here, run, run_c)
                return run + s_c, cc, run_c

            _, cc, run_c = lax.fori_loop(
                0, 16, chunk_body, (jnp.int32(0), jnp.int32(-1), jnp.int32(0))
            )

            acc_c = acc_chunk(cc)
            sfx = lax.rev(jnp.cumsum(lax.rev(acc_c, (0,))), (0,))
            m = (run_c + sfx) >= kk
            istar = jnp.sum(m.astype(jnp.int32)) - 1
            d_local = istar
            cnt_gt = run_c + jnp.sum(jnp.where(lane > istar, acc_c, 0))
            digit = cc * _L + d_local
            prefix_new = lax.shift_left(prefix, 8) | digit
            return prefix_new, kk - cnt_gt

        prefix, kk = radix_round(24, jnp.int32(0), k)
        prefix, kk = radix_round(16, prefix, kk)
        prefix, kk = radix_round(8, prefix, kk)
        t, take = radix_round(0, prefix, kk)

        lw_cp.wait()

        def final_sweep(i, carry):
            acc_gt, acc_eq, acc_neq = carry
            v = key_v[pl.ds(i * _L, _L)]
            w_ = lw_v[pl.ds(i * _L, _L)]
            mgt = v > t
            meq = v == t
            acc_gt = acc_gt + jnp.where(mgt, w_, 0.0)
            acc_eq = acc_eq + jnp.where(meq, w_, 0.0)
            acc_neq = acc_neq + meq.astype(jnp.int32)
            return acc_gt, acc_eq, acc_neq

        z = jnp.zeros((_L,), jnp.float32)
        acc_gt, acc_eq, acc_neq = lax.fori_loop(
            0, nchunks, final_sweep, (z, z, jnp.zeros((_L,), jnp.int32))
        )
        sg = jnp.broadcast_to(jnp.sum(acc_gt), (_L,))
        se = jnp.broadcast_to(jnp.sum(acc_eq), (_L,))
        ne = jnp.broadcast_to(jnp.sum(acc_neq), (_L,)).astype(jnp.float32)
        tk = jnp.broadcast_to(take, (_L,)).astype(jnp.float32)
        outv[...] = sg + tk * se / ne

    @pl.when(k <= 0)
    def _():
        lw_cp.wait()

    pltpu.sync_copy(outv, neg_hbm.at[wid])


def _combine_body(posw_ref, den_ref, negw_ref, out_ref):
    total = jnp.sum(posw_ref[:, :1]) + jnp.sum(negw_ref[:, :1])
    count = jnp.sum(den_ref[:, :1]).astype(jnp.float32)
    out_ref[...] = jnp.reshape(total / count, (1, 1))


def _sc_select(key, lw, kb):
    mesh = plsc.VectorSubcoreMesh(
        core_axis_name="c", subcore_axis_name="s", num_cores=_NC, num_subcores=_NS
    )
    return pl.kernel(
        _sc_select_body,
        out_type=jax.ShapeDtypeStruct((_ROWS, _L), jnp.float32),
        mesh=mesh,
        scratch_types=[
            pltpu.VMEM((_N,), jnp.int32),
            pltpu.VMEM((_N,), jnp.float32),
            pltpu.VMEM((_L,), jnp.int32),
            pltpu.VMEM((4096,), jnp.int32),
            pltpu.VMEM((_L,), jnp.float32),
            pltpu.SemaphoreType.DMA,
        ],
        compiler_params=pltpu.CompilerParams(needs_layout_passes=False),
    )(key, lw, kb)


@jax.jit
def kernel(pred, label):
    key, lw, kb, den, posw = pl.pallas_call(
        _prep_body,
        out_shape=[
            jax.ShapeDtypeStruct((_ROWS, _N), jnp.int32),
            jax.ShapeDtypeStruct((_ROWS, _N), jnp.float32),
            jax.ShapeDtypeStruct((_ROWS, _L), jnp.int32),
            jax.ShapeDtypeStruct((_ROWS, _L), jnp.int32),
            jax.ShapeDtypeStruct((_ROWS, _L), jnp.float32),
        ],
    )(pred, label)

    negw = _sc_select(key, lw, kb)

    out = pl.pallas_call(
        _combine_body,
        out_shape=jax.ShapeDtypeStruct((1, 1), jnp.float32),
    )(posw, den, negw)
    return out[0, 0]


# speedup vs baseline: 1.1174x; 1.1174x over previous
import functools
import jax
import jax.numpy as jnp
from jax import lax
from jax.experimental import pallas as pl
from jax.experimental.pallas import tpu as pltpu
from jax.experimental.pallas import tpu_sc as plsc

_GAMMA = 2.0
_ALPHA = 0.75
_NEG_RATIO = 3.0

_ROWS = 32
_N = 32768
_NC = 2
_NS = 16
_L = 16


def _prep_body(pred_ref, label_ref, key_ref, lw_ref, kb_ref, den_ref, posw_ref):
    pred = pred_ref[...]
    label = label_ref[...]
    n = pred.shape[1]

    e = jnp.exp(-jnp.abs(pred))
    log1pe = jnp.log(1.0 + e)
    ls_pos = jnp.minimum(pred, 0.0) - log1pe
    ls_neg = jnp.minimum(-pred, 0.0) - log1pe
    p = jnp.where(pred >= 0.0, 1.0 / (1.0 + e), e / (1.0 + e))

    loss = -(label * ls_pos + (1.0 - label) * ls_neg)
    p_t = label * p + (1.0 - label) * (1.0 - p)
    m = 1.0 - p_t
    loss = loss * (m * m)
    alpha_factor = label * _ALPHA + (1.0 - label) * (1.0 - _ALPHA)
    loss = loss * alpha_factor

    fn = (p < 0.5) & (label == 1.0)
    fp = (p >= 0.5) & (label == 0.0)
    w = _ALPHA / (1.0 - _ALPHA)
    loss_w = jnp.where(fn | fp, loss * w, loss)

    pos = label > 0.0
    num_pos = jnp.sum(pos.astype(jnp.int32), axis=1, keepdims=True)
    num_neg = (_NEG_RATIO * num_pos.astype(jnp.float32)).astype(jnp.int32)
    k = jnp.minimum(num_neg, n - num_pos)

    bits = lax.bitcast_convert_type(loss, jnp.int32)
    key_ref[...] = jnp.where(pos, 0, bits + 1)
    lw_ref[...] = loss_w
    kb_ref[...] = jnp.broadcast_to(k, (pred.shape[0], _L))
    den_ref[...] = jnp.broadcast_to(num_pos + k, (pred.shape[0], _L))
    posw_ref[...] = jnp.broadcast_to(
        jnp.sum(jnp.where(pos, loss_w, 0.0), axis=1, keepdims=True),
        (pred.shape[0], _L),
    )


def _sc_select_body(key_hbm, lw_hbm, kb_hbm, neg_hbm, key_v, lw_v, kv, hist, outv, sem):
    cid = lax.axis_index("c")
    sid = lax.axis_index("s")
    wid = sid * _NC + cid

    lw_cp = pltpu.async_copy(lw_hbm.at[wid], lw_v, sem)
    pltpu.sync_copy(key_hbm.at[wid], key_v)
    pltpu.sync_copy(kb_hbm.at[wid], kv)
    k = lax.shift_right_logical(jnp.sum(kv[...]), 4)

    outv[...] = jnp.zeros((_L,), jnp.float32)

    lane = lax.iota(jnp.int32, _L)
    ones = jnp.ones((_L,), jnp.int32)
    nchunks = _N // _L

    unroll = 8

    @pl.when(k > 0)
    def _():
        def radix_round(shift, prefix, kk):
            zero = jnp.zeros((_L,), jnp.int32)

            def zero_body(j, _):
                for u in range(unroll):
                    hist[pl.ds((j * unroll + u) * _L, _L)] = zero
                return 0

            lax.fori_loop(0, 4096 // _L // unroll, zero_body, 0)

            def sweep(i, _):
                for u in range(unroll):
                    v = key_v[pl.ds((i * unroll + u) * _L, _L)]
                    d = lax.shift_right_logical(v, shift) & 0xFF
                    idx = lane * 256 + d
                    if shift == 24:
                        plsc.addupdate_scatter(hist, [idx], ones)
                    else:
                        m = lax.shift_right_logical(v, shift + 8) == prefix
                        plsc.addupdate_scatter(hist, [idx], ones, mask=m)
                return 0

            lax.fori_loop(0, nchunks // unroll, sweep, 0)

            def acc_chunk(c):
                a = hist[pl.ds(c * _L, _L)]
                for l in range(1, _L):
                    a = a + hist[pl.ds((l * _L + c) * _L, _L)]
                return a

            def chunk_body(j, carry):
                run, cc, run_c = carry
                c = 15 - j
                s_c = jnp.sum(acc_chunk(c))
                here = jnp.logical_and(run + s_c >= kk, cc < 0)
                cc = jnp.where(here, c, cc)
                run_c = jnp.where(here, run, run_c)
                return run + s_c, cc, run_c

            _, cc, run_c = lax.fori_loop(
                0, 16, chunk_body, (jnp.int32(0), jnp.int32(-1), jnp.int32(0))
            )

            acc_c = acc_chunk(cc)
            sfx = lax.rev(jnp.cumsum(lax.rev(acc_c, (0,))), (0,))
            m = (run_c + sfx) >= kk
            istar = jnp.sum(m.astype(jnp.int32)) - 1
            d_local = istar
            cnt_gt = run_c + jnp.sum(jnp.where(lane > istar, acc_c, 0))
            digit = cc * _L + d_local
            prefix_new = lax.shift_left(prefix, 8) | digit
            return prefix_new, kk - cnt_gt

        prefix, kk = radix_round(24, jnp.int32(0), k)
        prefix, kk = radix_round(16, prefix, kk)
        prefix, kk = radix_round(8, prefix, kk)
        t, take = radix_round(0, prefix, kk)

        lw_cp.wait()

        def final_sweep(i, carry):
            acc_gt, acc_eq, acc_neq = carry
            for u in range(unroll):
                v = key_v[pl.ds((i * unroll + u) * _L, _L)]
                w_ = lw_v[pl.ds((i * unroll + u) * _L, _L)]
                mgt = v > t
                meq = v == t
                acc_gt = acc_gt + jnp.where(mgt, w_, 0.0)
                acc_eq = acc_eq + jnp.where(meq, w_, 0.0)
                acc_neq = acc_neq + meq.astype(jnp.int32)
            return acc_gt, acc_eq, acc_neq

        z = jnp.zeros((_L,), jnp.float32)
        acc_gt, acc_eq, acc_neq = lax.fori_loop(
            0, nchunks // unroll, final_sweep, (z, z, jnp.zeros((_L,), jnp.int32))
        )
        sg = jnp.broadcast_to(jnp.sum(acc_gt), (_L,))
        se = jnp.broadcast_to(jnp.sum(acc_eq), (_L,))
        ne = jnp.broadcast_to(jnp.sum(acc_neq), (_L,)).astype(jnp.float32)
        tk = jnp.broadcast_to(take, (_L,)).astype(jnp.float32)
        outv[...] = sg + tk * se / ne

    @pl.when(k <= 0)
    def _():
        lw_cp.wait()

    pltpu.sync_copy(outv, neg_hbm.at[wid])


def _combine_body(posw_ref, den_ref, negw_ref, out_ref):
    total = jnp.sum(posw_ref[:, :1]) + jnp.sum(negw_ref[:, :1])
    count = jnp.sum(den_ref[:, :1]).astype(jnp.float32)
    out_ref[...] = jnp.reshape(total / count, (1, 1))


def _sc_select(key, lw, kb):
    mesh = plsc.VectorSubcoreMesh(
        core_axis_name="c", subcore_axis_name="s", num_cores=_NC, num_subcores=_NS
    )
    return pl.kernel(
        _sc_select_body,
        out_type=jax.ShapeDtypeStruct((_ROWS, _L), jnp.float32),
        mesh=mesh,
        scratch_types=[
            pltpu.VMEM((_N,), jnp.int32),
            pltpu.VMEM((_N,), jnp.float32),
            pltpu.VMEM((_L,), jnp.int32),
            pltpu.VMEM((4096,), jnp.int32),
            pltpu.VMEM((_L,), jnp.float32),
            pltpu.SemaphoreType.DMA,
        ],
        compiler_params=pltpu.CompilerParams(needs_layout_passes=False),
    )(key, lw, kb)


@jax.jit
def kernel(pred, label):
    key, lw, kb, den, posw = pl.pallas_call(
        _prep_body,
        out_shape=[
            jax.ShapeDtypeStruct((_ROWS, _N), jnp.int32),
            jax.ShapeDtypeStruct((_ROWS, _N), jnp.float32),
            jax.ShapeDtypeStruct((_ROWS, _L), jnp.int32),
            jax.ShapeDtypeStruct((_ROWS, _L), jnp.int32),
            jax.ShapeDtypeStruct((_ROWS, _L), jnp.float32),
        ],
    )(pred, label)

    negw = _sc_select(key, lw, kb)

    out = pl.pallas_call(
        _combine_body,
        out_shape=jax.ShapeDtypeStruct((1, 1), jnp.float32),
    )(posw, den, negw)
    return out[0, 0]


# speedup vs baseline: 1.1927x; 1.0674x over previous
import functools
import jax
import jax.numpy as jnp
from jax import lax
from jax.experimental import pallas as pl
from jax.experimental.pallas import tpu as pltpu
from jax.experimental.pallas import tpu_sc as plsc

_GAMMA = 2.0
_ALPHA = 0.75
_NEG_RATIO = 3.0

_ROWS = 32
_N = 32768
_NC = 2
_NS = 16
_L = 16


def _prep_body(pred_ref, label_ref, key_ref, lw_ref, kb_ref, den_ref, posw_ref):
    pred = pred_ref[...]
    label = label_ref[...]
    n = pred.shape[1]

    e = jnp.exp(-jnp.abs(pred))
    log1pe = jnp.log(1.0 + e)
    ls_pos = jnp.minimum(pred, 0.0) - log1pe
    ls_neg = jnp.minimum(-pred, 0.0) - log1pe
    p = jnp.where(pred >= 0.0, 1.0 / (1.0 + e), e / (1.0 + e))

    loss = -(label * ls_pos + (1.0 - label) * ls_neg)
    p_t = label * p + (1.0 - label) * (1.0 - p)
    m = 1.0 - p_t
    loss = loss * (m * m)
    alpha_factor = label * _ALPHA + (1.0 - label) * (1.0 - _ALPHA)
    loss = loss * alpha_factor

    fn = (p < 0.5) & (label == 1.0)
    fp = (p >= 0.5) & (label == 0.0)
    w = _ALPHA / (1.0 - _ALPHA)
    loss_w = jnp.where(fn | fp, loss * w, loss)

    pos = label > 0.0
    num_pos = jnp.sum(pos.astype(jnp.int32), axis=1, keepdims=True)
    num_neg = (_NEG_RATIO * num_pos.astype(jnp.float32)).astype(jnp.int32)
    k = jnp.minimum(num_neg, n - num_pos)

    bits = lax.bitcast_convert_type(loss, jnp.int32)
    key_ref[...] = jnp.where(pos, 0, bits + 1)
    lw_ref[...] = loss_w
    kb_ref[...] = jnp.broadcast_to(k, (pred.shape[0], _L))
    den_ref[...] = jnp.broadcast_to(num_pos + k, (pred.shape[0], _L))
    posw_ref[...] = jnp.broadcast_to(
        jnp.sum(jnp.where(pos, loss_w, 0.0), axis=1, keepdims=True),
        (pred.shape[0], _L),
    )


def _sc_select_body(key_hbm, lw_hbm, kb_hbm, neg_hbm, key_v, lw_v, kv, hist, outv, sem):
    cid = lax.axis_index("c")
    sid = lax.axis_index("s")
    wid = sid * _NC + cid

    lw_cp = pltpu.async_copy(lw_hbm.at[wid], lw_v, sem)
    pltpu.sync_copy(key_hbm.at[wid], key_v)
    pltpu.sync_copy(kb_hbm.at[wid], kv)
    k = lax.shift_right_logical(jnp.sum(kv[...]), 4)

    outv[...] = jnp.zeros((_L,), jnp.float32)

    lane = lax.iota(jnp.int32, _L)
    ones = jnp.ones((_L,), jnp.int32)
    nchunks = _N // _L

    unroll = 8

    @pl.when(k > 0)
    def _():
        def radix_round(shift, prefix, kk):
            zero = jnp.zeros((_L,), jnp.int32)

            def zero_body(j, _):
                for u in range(unroll):
                    hist[pl.ds((j * unroll + u) * _L, _L)] = zero
                return 0

            lax.fori_loop(0, 4096 // _L // unroll, zero_body, 0)

            def sweep(i, _):
                for u in range(unroll):
                    v = key_v[pl.ds((i * unroll + u) * _L, _L)]
                    d = lax.shift_right_logical(v, shift) & 0xFF
                    idx = lax.shift_left(d, 4) + lane
                    if shift == 24:
                        plsc.addupdate_scatter(hist, [idx], ones)
                    else:
                        m = lax.shift_right_logical(v, shift + 8) == prefix
                        plsc.addupdate_scatter(hist, [idx], ones, mask=m)
                return 0

            lax.fori_loop(0, nchunks // unroll, sweep, 0)

            def chunk_sum(c):
                a = hist[pl.ds(c * 256, _L)]
                for j in range(1, _L):
                    a = a + hist[pl.ds(c * 256 + j * _L, _L)]
                return jnp.sum(a)

            def chunk_body(j, carry):
                run, cc, run_c = carry
                c = 15 - j
                s_c = chunk_sum(c)
                here = jnp.logical_and(run + s_c >= kk, cc < 0)
                cc = jnp.where(here, c, cc)
                run_c = jnp.where(here, run, run_c)
                return run + s_c, cc, run_c

            _, cc, run_c = lax.fori_loop(
                0, 16, chunk_body, (jnp.int32(0), jnp.int32(-1), jnp.int32(0))
            )

            accs = [
                jnp.sum(hist[pl.ds(cc * 256 + i * _L, _L)]) for i in range(_L)
            ]
            sfx = [None] * _L
            s = jnp.int32(0)
            for i in range(_L - 1, -1, -1):
                s = s + accs[i]
                sfx[i] = s
            istar = sum(
                [(run_c + sfx[i] >= kk).astype(jnp.int32) for i in range(_L)]
            ) - 1
            above = jnp.int32(0)
            for i in range(_L):
                above = above + jnp.where(i > istar, accs[i], 0)
            cnt_gt = run_c + above
            digit = cc * _L + istar
            prefix_new = lax.shift_left(prefix, 8) | digit
            return prefix_new, kk - cnt_gt

        prefix, kk = radix_round(24, jnp.int32(0), k)
        prefix, kk = radix_round(16, prefix, kk)
        prefix, kk = radix_round(8, prefix, kk)
        t, take = radix_round(0, prefix, kk)

        lw_cp.wait()

        def final_sweep(i, carry):
            acc_gt, acc_eq, acc_neq = carry
            for u in range(unroll):
                v = key_v[pl.ds((i * unroll + u) * _L, _L)]
                w_ = lw_v[pl.ds((i * unroll + u) * _L, _L)]
                mgt = v > t
                meq = v == t
                acc_gt = acc_gt + jnp.where(mgt, w_, 0.0)
                acc_eq = acc_eq + jnp.where(meq, w_, 0.0)
                acc_neq = acc_neq + meq.astype(jnp.int32)
            return acc_gt, acc_eq, acc_neq

        z = jnp.zeros((_L,), jnp.float32)
        acc_gt, acc_eq, acc_neq = lax.fori_loop(
            0, nchunks // unroll, final_sweep, (z, z, jnp.zeros((_L,), jnp.int32))
        )
        sg = jnp.broadcast_to(jnp.sum(acc_gt), (_L,))
        se = jnp.broadcast_to(jnp.sum(acc_eq), (_L,))
        ne = jnp.broadcast_to(jnp.sum(acc_neq), (_L,)).astype(jnp.float32)
        tk = jnp.broadcast_to(take, (_L,)).astype(jnp.float32)
        outv[...] = sg + tk * se / ne

    @pl.when(k <= 0)
    def _():
        lw_cp.wait()

    pltpu.sync_copy(outv, neg_hbm.at[wid])


def _combine_body(posw_ref, den_ref, negw_ref, out_ref):
    total = jnp.sum(posw_ref[:, :1]) + jnp.sum(negw_ref[:, :1])
    count = jnp.sum(den_ref[:, :1]).astype(jnp.float32)
    out_ref[...] = jnp.reshape(total / count, (1, 1))


def _sc_select(key, lw, kb):
    mesh = plsc.VectorSubcoreMesh(
        core_axis_name="c", subcore_axis_name="s", num_cores=_NC, num_subcores=_NS
    )
    return pl.kernel(
        _sc_select_body,
        out_type=jax.ShapeDtypeStruct((_ROWS, _L), jnp.float32),
        mesh=mesh,
        scratch_types=[
            pltpu.VMEM((_N,), jnp.int32),
            pltpu.VMEM((_N,), jnp.float32),
            pltpu.VMEM((_L,), jnp.int32),
            pltpu.VMEM((4096,), jnp.int32),
            pltpu.VMEM((_L,), jnp.float32),
            pltpu.SemaphoreType.DMA,
        ],
        compiler_params=pltpu.CompilerParams(needs_layout_passes=False),
    )(key, lw, kb)


@jax.jit
def kernel(pred, label):
    key, lw, kb, den, posw = pl.pallas_call(
        _prep_body,
        out_shape=[
            jax.ShapeDtypeStruct((_ROWS, _N), jnp.int32),
            jax.ShapeDtypeStruct((_ROWS, _N), jnp.float32),
            jax.ShapeDtypeStruct((_ROWS, _L), jnp.int32),
            jax.ShapeDtypeStruct((_ROWS, _L), jnp.int32),
            jax.ShapeDtypeStruct((_ROWS, _L), jnp.float32),
        ],
    )(pred, label)

    negw = _sc_select(key, lw, kb)

    out = pl.pallas_call(
        _combine_body,
        out_shape=jax.ShapeDtypeStruct((1, 1), jnp.float32),
    )(posw, den, negw)
    return out[0, 0]


# speedup vs baseline: 2.3372x; 1.9595x over previous
import functools
import jax
import jax.numpy as jnp
from jax import lax
from jax.experimental import pallas as pl
from jax.experimental.pallas import tpu as pltpu
from jax.experimental.pallas import tpu_sc as plsc

_GAMMA = 2.0
_ALPHA = 0.75
_NEG_RATIO = 3.0

_ROWS = 32
_N = 32768
_NC = 2
_NS = 16
_L = 16


def _prep_body(pred_ref, label_ref, key_ref, lw_ref, kb_ref, den_ref, posw_ref):
    pred = pred_ref[...]
    label = label_ref[...]
    n = pred.shape[1]

    e = jnp.exp(-jnp.abs(pred))
    log1pe = jnp.log(1.0 + e)
    ls_pos = jnp.minimum(pred, 0.0) - log1pe
    ls_neg = jnp.minimum(-pred, 0.0) - log1pe
    p = jnp.where(pred >= 0.0, 1.0 / (1.0 + e), e / (1.0 + e))

    loss = -(label * ls_pos + (1.0 - label) * ls_neg)
    p_t = label * p + (1.0 - label) * (1.0 - p)
    m = 1.0 - p_t
    loss = loss * (m * m)
    alpha_factor = label * _ALPHA + (1.0 - label) * (1.0 - _ALPHA)
    loss = loss * alpha_factor

    fn = (p < 0.5) & (label == 1.0)
    fp = (p >= 0.5) & (label == 0.0)
    w = _ALPHA / (1.0 - _ALPHA)
    loss_w = jnp.where(fn | fp, loss * w, loss)

    pos = label > 0.0
    num_pos = jnp.sum(pos.astype(jnp.int32), axis=1, keepdims=True)
    num_neg = (_NEG_RATIO * num_pos.astype(jnp.float32)).astype(jnp.int32)
    k = jnp.minimum(num_neg, n - num_pos)

    bits = lax.bitcast_convert_type(loss, jnp.int32)
    key_ref[...] = jnp.where(pos, 0, bits + 1)
    lw_ref[...] = loss_w
    kb_ref[...] = jnp.broadcast_to(k, (pred.shape[0], _L))
    den_ref[...] = jnp.broadcast_to(num_pos + k, (pred.shape[0], _L))
    posw_ref[...] = jnp.broadcast_to(
        jnp.sum(jnp.where(pos, loss_w, 0.0), axis=1, keepdims=True),
        (pred.shape[0], _L),
    )


def _sc_select_body(key_hbm, lw_hbm, kb_hbm, neg_hbm, key_v, lw_v, kv, hist, outv, sem):
    cid = lax.axis_index("c")
    sid = lax.axis_index("s")
    wid = sid * _NC + cid

    lw_cp = pltpu.async_copy(lw_hbm.at[wid], lw_v, sem)
    pltpu.sync_copy(key_hbm.at[wid], key_v)
    pltpu.sync_copy(kb_hbm.at[wid], kv)
    k = lax.shift_right_logical(jnp.sum(kv[...]), 4)

    outv[...] = jnp.zeros((_L,), jnp.float32)

    lane = lax.iota(jnp.int32, _L)
    ones = jnp.ones((_L,), jnp.int32)
    nchunks = _N // _L

    unroll = 8

    @pl.when(k > 0)
    def _():
        def radix_round(shift, prefix, kk):
            zero = jnp.zeros((_L,), jnp.int32)

            @plsc.parallel_loop(0, 4096 // _L, unroll=unroll)
            def _(j):
                hist[pl.ds(j * _L, _L)] = zero

            @plsc.parallel_loop(0, nchunks, unroll=unroll)
            def _(i):
                v = key_v[pl.ds(i * _L, _L)]
                d = lax.shift_right_logical(v, shift) & 0xFF
                idx = lax.shift_left(d, 4) + lane
                if shift == 24:
                    plsc.addupdate_scatter(hist, [idx], ones)
                else:
                    m = lax.shift_right_logical(v, shift + 8) == prefix
                    plsc.addupdate_scatter(hist, [idx], ones, mask=m)

            def chunk_sum(c):
                a = hist[pl.ds(c * 256, _L)]
                for j in range(1, _L):
                    a = a + hist[pl.ds(c * 256 + j * _L, _L)]
                return jnp.sum(a)

            def chunk_body(j, carry):
                run, cc, run_c = carry
                c = 15 - j
                s_c = chunk_sum(c)
                here = jnp.logical_and(run + s_c >= kk, cc < 0)
                cc = jnp.where(here, c, cc)
                run_c = jnp.where(here, run, run_c)
                return run + s_c, cc, run_c

            _, cc, run_c = lax.fori_loop(
                0, 16, chunk_body, (jnp.int32(0), jnp.int32(-1), jnp.int32(0))
            )

            accs = [
                jnp.sum(hist[pl.ds(cc * 256 + i * _L, _L)]) for i in range(_L)
            ]
            sfx = [None] * _L
            s = jnp.int32(0)
            for i in range(_L - 1, -1, -1):
                s = s + accs[i]
                sfx[i] = s
            istar = sum(
                [(run_c + sfx[i] >= kk).astype(jnp.int32) for i in range(_L)]
            ) - 1
            above = jnp.int32(0)
            for i in range(_L):
                above = above + jnp.where(i > istar, accs[i], 0)
            cnt_gt = run_c + above
            digit = cc * _L + istar
            prefix_new = lax.shift_left(prefix, 8) | digit
            return prefix_new, kk - cnt_gt

        prefix, kk = radix_round(24, jnp.int32(0), k)
        prefix, kk = radix_round(16, prefix, kk)
        prefix, kk = radix_round(8, prefix, kk)
        t, take = radix_round(0, prefix, kk)

        lw_cp.wait()

        z = jnp.zeros((_L,), jnp.float32)

        @plsc.parallel_loop(
            0, nchunks, unroll=unroll, carry=(z, z, jnp.zeros((_L,), jnp.int32))
        )
        def acc_out(i, carry):
            acc_gt, acc_eq, acc_neq = carry
            v = key_v[pl.ds(i * _L, _L)]
            w_ = lw_v[pl.ds(i * _L, _L)]
            mgt = v > t
            meq = v == t
            acc_gt = acc_gt + jnp.where(mgt, w_, 0.0)
            acc_eq = acc_eq + jnp.where(meq, w_, 0.0)
            acc_neq = acc_neq + meq.astype(jnp.int32)
            return acc_gt, acc_eq, acc_neq

        acc_gt, acc_eq, acc_neq = acc_out
        sg = jnp.broadcast_to(jnp.sum(acc_gt), (_L,))
        se = jnp.broadcast_to(jnp.sum(acc_eq), (_L,))
        ne = jnp.broadcast_to(jnp.sum(acc_neq), (_L,)).astype(jnp.float32)
        tk = jnp.broadcast_to(take, (_L,)).astype(jnp.float32)
        outv[...] = sg + tk * se / ne

    @pl.when(k <= 0)
    def _():
        lw_cp.wait()

    pltpu.sync_copy(outv, neg_hbm.at[wid])


def _combine_body(posw_ref, den_ref, negw_ref, out_ref):
    total = jnp.sum(posw_ref[:, :1]) + jnp.sum(negw_ref[:, :1])
    count = jnp.sum(den_ref[:, :1]).astype(jnp.float32)
    out_ref[...] = jnp.reshape(total / count, (1, 1))


def _sc_select(key, lw, kb):
    mesh = plsc.VectorSubcoreMesh(
        core_axis_name="c", subcore_axis_name="s", num_cores=_NC, num_subcores=_NS
    )
    return pl.kernel(
        _sc_select_body,
        out_type=jax.ShapeDtypeStruct((_ROWS, _L), jnp.float32),
        mesh=mesh,
        scratch_types=[
            pltpu.VMEM((_N,), jnp.int32),
            pltpu.VMEM((_N,), jnp.float32),
            pltpu.VMEM((_L,), jnp.int32),
            pltpu.VMEM((4096,), jnp.int32),
            pltpu.VMEM((_L,), jnp.float32),
            pltpu.SemaphoreType.DMA,
        ],
        compiler_params=pltpu.CompilerParams(needs_layout_passes=False),
    )(key, lw, kb)


@jax.jit
def kernel(pred, label):
    key, lw, kb, den, posw = pl.pallas_call(
        _prep_body,
        out_shape=[
            jax.ShapeDtypeStruct((_ROWS, _N), jnp.int32),
            jax.ShapeDtypeStruct((_ROWS, _N), jnp.float32),
            jax.ShapeDtypeStruct((_ROWS, _L), jnp.int32),
            jax.ShapeDtypeStruct((_ROWS, _L), jnp.int32),
            jax.ShapeDtypeStruct((_ROWS, _L), jnp.float32),
        ],
    )(pred, label)

    negw = _sc_select(key, lw, kb)

    out = pl.pallas_call(
        _combine_body,
        out_shape=jax.ShapeDtypeStruct((1, 1), jnp.float32),
    )(posw, den, negw)
    return out[0, 0]
